# Initial kernel scaffold; baseline (speedup 1.0000x reference)
import jax
import jax.numpy as jnp
from jax.experimental import pallas as pl


def kernel(edge_index, f, d, a, w1, w2, W1, b1, W2, b2, W3, b3):
    raise NotImplementedError("write your pallas kernel here")


# speedup vs baseline: 14.6454x; 14.6454x over previous
import jax
import jax.numpy as jnp
from jax import lax
from jax.experimental import pallas as pl
from jax.experimental.pallas import tpu as pltpu
from jax.experimental.pallas import tpu_sc as plsc

N = 10000
E = 320000
D = 129
L = 16
DP = 144
NB = DP // L
NC = 2
NS = 16
EPW = E // (NC * NS)
C = 80
NCH = EPW // C
RPT = 624
RF = RPT // C
RR = RPT - RF * C
TAILR = N - NS * RPT


def _tanh(x):
    return 1.0 - 2.0 / (jnp.exp(x + x) + 1.0)


def _sc_body(f_hbm, src_hbm, tgt_hbm, a_hbm, d_hbm, w1_hbm, w2_hbm, agg_hbm,
             idx_s, idx_t, a_v, d_v, ua_v, w1_v, w2_v,
             rows_s, rows_t, msg, agg_sh, sem_s, sem_t):
    cid = lax.axis_index("c")
    sid = lax.axis_index("s")
    base = (cid * NS + sid) * EPW

    pltpu.sync_copy(w1_hbm, w1_v)
    pltpu.sync_copy(w2_hbm, w2_v)
    w1r = w1_v[...]
    w2r = w2_v[...]

    zero = jnp.zeros((L,), jnp.float32)

    def zrow(r, carry):
        for b in range(NB):
            msg[r, pl.ds(b * L, L)] = zero
        return carry

    lax.fori_loop(0, C, zrow, 0)

    row0 = pl.multiple_of(sid * RPT, 8)

    def zcp(k, carry):
        pltpu.sync_copy(msg, agg_sh.at[pl.ds(pl.multiple_of(row0 + k * C, 8), C)])
        return carry

    lax.fori_loop(0, RF, zcp, 0)
    pltpu.sync_copy(msg.at[pl.ds(0, RR)],
                    agg_sh.at[pl.ds(pl.multiple_of(row0 + RF * C, 8), RR)])

    @pl.when(sid == NS - 1)
    def _():
        pltpu.sync_copy(msg.at[pl.ds(0, TAILR)],
                        agg_sh.at[pl.ds(N - TAILR, TAILR)])

    plsc.subcore_barrier()

    lane = lax.iota(jnp.int32, L)

    def chunk(k, carry):
        e0 = pl.multiple_of(base + k * C, 8)
        pltpu.sync_copy(src_hbm.at[pl.ds(e0, C)], idx_s)
        pltpu.sync_copy(tgt_hbm.at[pl.ds(e0, C)], idx_t)
        pltpu.sync_copy(a_hbm.at[pl.ds(e0, C)], a_v)
        pltpu.sync_copy(d_hbm.at[pl.ds(e0, C)], d_v.at[pl.ds(0, C)])
        cs = pltpu.async_copy(f_hbm.at[idx_s], rows_s, sem_s)
        ct = pltpu.async_copy(f_hbm.at[idx_t], rows_t, sem_t)
        for i in range(C // L):
            ua_v[pl.ds(i * L, L)] = w1r * a_v[pl.ds(i * L, L)]
        cs.wait()
        ct.wait()

        def edge(e, ecarry):
            ua = jnp.full((L,), ua_v[pl.ds(e, L)][0], jnp.float32)
            de = jnp.full((L,), d_v[pl.ds(e, L)][0], jnp.float32)
            for b in range(NB):
                xs = rows_s[e, pl.ds(b * L, L)]
                xt = rows_t[e, pl.ds(b * L, L)]
                m = _tanh(w2r * _tanh(ua * xs)) + _tanh(w2r * _tanh(ua * xt))
                if b == NB - 1:
                    m = jnp.where(lane == (D - (NB - 1) * L), de, m)
                    m = jnp.where(lane == (D + 1 - (NB - 1) * L),
                                  jnp.float32(1.0), m)
                msg[e, pl.ds(b * L, L)] = m
            return ecarry

        lax.fori_loop(0, C, edge, 0)
        pltpu.sync_copy(msg, agg_sh.at[idx_t], add=True)
        return carry

    lax.fori_loop(0, NCH, chunk, 0)
    plsc.subcore_barrier()

    def wout(k, carry):
        r = pl.multiple_of(row0 + k * C, 8)
        pltpu.sync_copy(agg_sh.at[pl.ds(r, C)], agg_hbm.at[cid].at[pl.ds(r, C)])
        return carry

    lax.fori_loop(0, RF, wout, 0)
    rlast = pl.multiple_of(row0 + RF * C, 8)
    pltpu.sync_copy(agg_sh.at[pl.ds(rlast, RR)],
                    agg_hbm.at[cid].at[pl.ds(rlast, RR)])

    @pl.when(sid == NS - 1)
    def _():
        pltpu.sync_copy(agg_sh.at[pl.ds(N - TAILR, TAILR)],
                        agg_hbm.at[cid].at[pl.ds(N - TAILR, TAILR)])


_sc_call = pl.kernel(
    _sc_body,
    out_type=jax.ShapeDtypeStruct((NC, N, DP), jnp.float32),
    mesh=plsc.VectorSubcoreMesh(core_axis_name="c", subcore_axis_name="s"),
    compiler_params=pltpu.CompilerParams(use_tc_tiling_on_sc=False),
    scratch_types=[
        pltpu.VMEM((C,), jnp.int32),
        pltpu.VMEM((C,), jnp.int32),
        pltpu.VMEM((C,), jnp.float32),
        pltpu.VMEM((C + L,), jnp.float32),
        pltpu.VMEM((C + L,), jnp.float32),
        pltpu.VMEM((L,), jnp.float32),
        pltpu.VMEM((L,), jnp.float32),
        pltpu.VMEM((C, DP), jnp.float32),
        pltpu.VMEM((C, DP), jnp.float32),
        pltpu.VMEM((C, DP), jnp.float32),
        pltpu.VMEM_SHARED((N, DP), jnp.float32),
        pltpu.SemaphoreType.DMA,
        pltpu.SemaphoreType.DMA,
    ],
)


BROWS = 2000


def _tc_body(f_ref, p0_ref, p1_ref, W1_ref, b1_ref, W2_ref, b2_ref,
             W3_ref, b3_ref, o_ref):
    agg = p0_ref[...] + p1_ref[...]
    col = lax.broadcasted_iota(jnp.int32, agg.shape, 1)
    aggm = jnp.where(col < D, agg, 0.0)
    nd = agg[:, D:D + 1]
    ncnt = agg[:, D + 1:D + 2]
    f = f_ref[...]
    f_inv = jnp.sqrt(jnp.sum(f * f, axis=1, keepdims=True))
    msg_inv = jnp.sqrt(jnp.sum(aggm * aggm, axis=1, keepdims=True))
    avg = nd / (ncnt + 1e-8)
    psi = jnp.concatenate([f_inv, msg_inv, avg], axis=1)
    h = jax.nn.relu(jnp.dot(psi, W1_ref[...].T,
                            preferred_element_type=jnp.float32) + b1_ref[...])
    h = jax.nn.relu(jnp.dot(h, W2_ref[...].T,
                            preferred_element_type=jnp.float32) + b2_ref[...])
    gate = jax.nn.sigmoid(jnp.sum(h * W3_ref[...], axis=1, keepdims=True)
                          + b3_ref[0, 0])
    o_ref[...] = f + gate * aggm


def _tc_call(fpad, p0, p1, W1, b1, W2, b2, W3, b3):
    full = lambda shape: pl.BlockSpec(shape, lambda i: (0, 0))
    return pl.pallas_call(
        _tc_body,
        grid=(N // BROWS,),
        in_specs=[
            pl.BlockSpec((BROWS, DP), lambda i: (i, 0)),
            pl.BlockSpec((BROWS, DP), lambda i: (i, 0)),
            pl.BlockSpec((BROWS, DP), lambda i: (i, 0)),
            full((64, 3)), full((1, 64)),
            full((32, 64)), full((1, 32)),
            full((1, 32)),
            pl.BlockSpec(memory_space=pltpu.SMEM),
        ],
        out_specs=pl.BlockSpec((BROWS, DP), lambda i: (i, 0)),
        out_shape=jax.ShapeDtypeStruct((N, DP), jnp.float32),
    )(fpad, p0, p1, W1, b1, W2, b2, W3, b3)


@jax.jit
def kernel(edge_index, f, d, a, w1, w2, W1, b1, W2, b2, W3, b3):
    src = edge_index[0].astype(jnp.int32)
    tgt = edge_index[1].astype(jnp.int32)
    fpad = jnp.pad(f, ((0, 0), (0, DP - D)))
    a1 = a[:, 0]
    d1 = d[:, 0]
    w1b = jnp.full((L,), w1[0], jnp.float32)
    w2b = jnp.full((L,), w2[0], jnp.float32)
    aggp = _sc_call(fpad, src, tgt, a1, d1, w1b, w2b)
    outp = _tc_call(fpad, aggp[0], aggp[1], W1, b1.reshape(1, 64),
                    W2, b2.reshape(1, 32), W3, b3.reshape(1, 1))
    return outp[:, :D]
